# Initial kernel scaffold; baseline (speedup 1.0000x reference)
import jax
import jax.numpy as jnp
from jax.experimental import pallas as pl


def kernel(history_items, pool_tokens, pool_phis, embed_table, fc_w, fc_b):
    raise NotImplementedError("write your pallas kernel here")


# speedup vs baseline: 3.3369x; 3.3369x over previous
import functools

import numpy as np
import jax
import jax.numpy as jnp
from jax import lax
from jax.experimental import pallas as pl
from jax.experimental.pallas import tpu as pltpu
from jax.experimental.pallas import tpu_sc as plsc

B = 16384
H = 50
D = 32
V = 256
P = 1024
S = 3
LEVELS = 3
ETA = 1.0
TINY = np.float32(np.finfo(np.float32).tiny)

NC = 2
NS = 16
NW = NC * NS
B_PER_W = B // NW
IDX_CHUNKS = B_PER_W // 128


def _np_threefry2x32(k1, k2, x0, x1):
    def rotl(x, d):
        return ((x << np.uint32(d)) | (x >> np.uint32(32 - d))).astype(np.uint32)
    ks = [np.uint32(k1), np.uint32(k2),
          np.uint32(np.uint32(k1) ^ np.uint32(k2) ^ np.uint32(0x1BD11BDA))]
    x = [(x0 + ks[0]).astype(np.uint32), (x1 + ks[1]).astype(np.uint32)]
    rot = ((13, 15, 26, 6), (17, 29, 16, 24))
    for i in range(5):
        for r in rot[i % 2]:
            x[0] = (x[0] + x[1]).astype(np.uint32)
            x[1] = rotl(x[1], r)
            x[1] = (x[0] ^ x[1]).astype(np.uint32)
        x[0] = (x[0] + ks[(i + 1) % 3]).astype(np.uint32)
        x[1] = (x[1] + ks[(i + 2) % 3] + np.uint32(i + 1)).astype(np.uint32)
    return x[0], x[1]


def _fold_key(level):
    o0, o1 = _np_threefry2x32(np.uint32(0), np.uint32(42),
                              np.array([0], np.uint32),
                              np.array([level], np.uint32))
    return int(o0[0]), int(o1[0])


_KEYS = tuple(_fold_key(level) for level in range(LEVELS))


def _sc_body(table_hbm, idx_hbm, ctx_hbm, idx_v, rows_v, sem):
    wid = lax.axis_index("s") * NC + lax.axis_index("c")
    base = wid * B_PER_W

    pltpu.sync_copy(idx_hbm.at[wid], idx_v)
    copies = []
    for j in range(IDX_CHUNKS):
        copies.append(pltpu.async_copy(
            table_hbm.at[idx_v.at[j]], rows_v.at[pl.ds(j * 128, 128)], sem))
    for c in copies:
        c.wait()
    pltpu.sync_copy(rows_v, ctx_hbm.at[pl.ds(base, B_PER_W)])


@functools.cache
def _sc_gather_fn():
    return functools.partial(
        pl.kernel,
        mesh=plsc.VectorSubcoreMesh(core_axis_name="c", subcore_axis_name="s"),
        compiler_params=pltpu.CompilerParams(use_tc_tiling_on_sc=False),
        out_type=[
            jax.ShapeDtypeStruct((B, D), jnp.float32),
        ],
        scratch_types=[
            pltpu.VMEM((IDX_CHUNKS, 128), jnp.int32),
            pltpu.VMEM((B_PER_W, D), jnp.float32),
            pltpu.SemaphoreType.DMA,
        ],
    )(_sc_body)


def _rotl(x, d):
    return lax.shift_left(x, jnp.uint32(d)) | lax.shift_right_logical(
        x, jnp.uint32(32 - d))


def _threefry_bits(k1, k2, n_u32):
    ks0 = np.uint32(k1)
    ks1 = np.uint32(k2)
    ks2 = np.uint32(ks0 ^ ks1 ^ np.uint32(0x1BD11BDA))
    ks = (ks0, ks1, ks2)
    x0 = jnp.full_like(n_u32, ks0)
    x1 = n_u32 + ks1
    rot = ((13, 15, 26, 6), (17, 29, 16, 24))
    for i in range(5):
        for r in rot[i % 2]:
            x0 = x0 + x1
            x1 = _rotl(x1, r)
            x1 = x0 ^ x1
        x0 = x0 + ks[(i + 1) % 3]
        x1 = x1 + np.uint32(ks[(i + 2) % 3] + np.uint32(i + 1))
    return x0 ^ x1


def _bits_to_uniform(bits):
    fb = lax.shift_right_logical(bits, jnp.uint32(9)) | jnp.uint32(0x3F800000)
    f = lax.bitcast_convert_type(fb, jnp.float32) - jnp.float32(1.0)
    return jnp.maximum(TINY, f + TINY)


R = 512


def _tc_body(ctx_ref, tok_ref, phi_ref, fcw_ref, fcb_ref, out_ref,
             w_scr, pen0_scr):
    @pl.when(pl.program_id(0) == 0)
    def _():
        phi = phi_ref[...]
        pen0_scr[0] = jnp.sum(phi)
        pv_ids = lax.broadcasted_iota(jnp.int32, (P, V), 1)
        for level in (1, 2):
            eq = tok_ref[:, level - 1:level] == pv_ids
            w_scr[level - 1:level, :] = jnp.sum(
                jnp.where(eq, phi, jnp.float32(0.0)), axis=0, keepdims=True)

    i0 = pl.program_id(0) * R
    ctx = ctx_ref[...]
    logits = jnp.dot(ctx, fcw_ref[...],
                     preferred_element_type=jnp.float32) + fcb_ref[...]
    row_ids = i0 + lax.broadcasted_iota(jnp.int32, (R, V), 0)
    v_ids = lax.broadcasted_iota(jnp.int32, (R, V), 1)
    base_n = (row_ids * V + v_ids).astype(jnp.uint32)
    for level in range(LEVELS):
        if level == 0:
            x = logits - ETA * pen0_scr[0]
        else:
            pen = jnp.sum(logits * w_scr[level - 1:level, :],
                          axis=-1, keepdims=True)
            x = logits - ETA * pen
        k1, k2 = _KEYS[level]
        for s in range(S):
            n = base_n + np.uint32(s * B * V)
            bits = _threefry_bits(k1, k2, n)
            u = _bits_to_uniform(bits)
            g = -jnp.log(-jnp.log(u))
            r = g + x
            m = jnp.max(r, axis=-1, keepdims=True)
            tok = jnp.min(jnp.where(r == m, v_ids, V), axis=-1, keepdims=True)
            c = s * LEVELS + level
            out_ref[:, c:c + 1] = tok


def _tc_sample(ctx, tok12, phi_col, fc_w, fc_b):
    return pl.pallas_call(
        _tc_body,
        grid=(B // R,),
        in_specs=[
            pl.BlockSpec((R, D), lambda i: (i, 0)),
            pl.BlockSpec((P, 2), lambda i: (0, 0)),
            pl.BlockSpec((P, 1), lambda i: (0, 0)),
            pl.BlockSpec((D, V), lambda i: (0, 0)),
            pl.BlockSpec((1, V), lambda i: (0, 0)),
        ],
        out_specs=pl.BlockSpec((R, 128), lambda i: (i, 0)),
        out_shape=jax.ShapeDtypeStruct((B, 128), jnp.int32),
        scratch_shapes=[
            pltpu.VMEM((2, V), jnp.float32),
            pltpu.SMEM((1,), jnp.float32),
        ],
    )(ctx, tok12, phi_col, fc_w, fc_b)


def kernel(history_items, pool_tokens, pool_phis, embed_table, fc_w, fc_b):
    last_idx = history_items[:, -1].reshape(NW, IDX_CHUNKS, 128)
    (ctx,) = _sc_gather_fn()(embed_table, last_idx)
    out = _tc_sample(ctx, pool_tokens[:, 1:3], pool_phis.reshape(P, 1),
                     fc_w, fc_b.reshape(1, V))
    return out[:, :S * LEVELS].reshape(B, S, LEVELS)
